# baseline (speedup 1.0000x reference)
import functools

import jax
import jax.numpy as jnp
from jax import lax
from jax.experimental import pallas as pl
from jax.experimental.pallas import tpu as pltpu
from jax.experimental.pallas import tpu_sc as plsc

_BATCH = 16384
_HIST = 200
_EMBED = 32
_N = _BATCH * _HIST

_NC = 2
_NS = 16
_NW = _NC * _NS
_MB = 128
_SB = 512
_NSB = _N // _SB
_BPW = _NSB // _NW
_SBH = _BATCH // _SB

_mesh = plsc.VectorSubcoreMesh(core_axis_name="c", subcore_axis_name="s")


@functools.partial(
    pl.kernel,
    out_type=jax.ShapeDtypeStruct((_HIST, 4, _BATCH * 8), jnp.float32),
    mesh=_mesh,
    scratch_types=[
        pltpu.VMEM((_SB,), jnp.int32),
        pltpu.VMEM((_SB,), jnp.int32),
        pltpu.VMEM((_SB, _EMBED), jnp.float32),
        pltpu.VMEM((_SB, _EMBED), jnp.float32),
        pltpu.VMEM((4, 4096), jnp.float32),
        pltpu.VMEM((4, 4096), jnp.float32),
        pltpu.SemaphoreType.DMA,
        pltpu.SemaphoreType.DMA,
        pltpu.SemaphoreType.DMA,
        pltpu.SemaphoreType.DMA,
        pltpu.SemaphoreType.DMA,
        pltpu.SemaphoreType.DMA,
    ],
    compiler_params=pltpu.CompilerParams(use_tc_tiling_on_sc=False,
                                         needs_layout_passes=False),
)
def _gather_kernel(xt_hbm, table_hbm, out_hbm, idx_v0, idx_v1,
                   rows_v0, rows_v1, buf_v0, buf_v1,
                   sem_l0, sem_l1, sem_g0, sem_g1, sem_w0, sem_w1):
    wid = lax.axis_index("s") * _NC + lax.axis_index("c")
    t0 = wid * _BPW
    idx_v = (idx_v0, idx_v1)
    rows_v = (rows_v0, rows_v1)
    buf_v = (buf_v0, buf_v1)
    sem_l = (sem_l0, sem_l1)
    sem_g = (sem_g0, sem_g1)
    sem_w = (sem_w0, sem_w1)

    iota = lax.iota(jnp.int32, 16)

    def hB(l):
        t = t0 + l
        return t // _SBH, t % _SBH

    def l_copy(l, b):
        h, B = hB(l)
        return pltpu.make_async_copy(
            xt_hbm.at[pl.ds(h * _BATCH + B * _SB, _SB)], idx_v[b], sem_l[b])

    def g_copy(b):
        return pltpu.make_async_copy(
            table_hbm.at[idx_v[b]], rows_v[b], sem_g[b])

    def _w_copy(l, b):
        h, B = hB(l)
        return pltpu.make_async_copy(
            buf_v[b], out_hbm.at[h, :, pl.ds(B * 4096, 4096)], sem_w[b])

    def w_start(l, b):
        _w_copy(l, b).start()

    def w_wait(l, b):
        _w_copy(l, b).wait()

    def transpose(b):
        def emit(c0):
            cm = (c0 + iota) & 31
            q_idx = cm >> 3
            sbase = ((cm & 7) << 7) + iota

            def load(k):
                return plsc.load_gather(rows_v[b], [iota + 16 * k, cm])

            def store(k, v):
                off_k = 1024 * (k // 8) + (16 * k % 128)
                plsc.store_scatter(buf_v[b], [q_idx, sbase + off_k], v)

            depth = 4
            pend = [load(k) for k in range(depth)]
            for k in range(32 - depth):
                nxt = load(k + depth)
                store(k, pend[0])
                pend = pend[1:] + [nxt]
            for k in range(32 - depth, 32):
                store(k, pend[0])
                pend = pend[1:]

        def step(i, carry):
            emit(2 * i)
            emit(2 * i + 1)
            return carry

        lax.fori_loop(0, 16, step, 0)

    l_copy(0, 0).start()
    l_copy(1, 1).start()
    l_copy(0, 0).wait()
    g_copy(0).start()
    g_copy(0).wait()
    l_copy(1, 1).wait()
    g_copy(1).start()
    transpose(0)
    w_start(0, 0)
    l_copy(2, 0).start()
    g_copy(1).wait()
    l_copy(2, 0).wait()
    g_copy(0).start()
    transpose(1)
    w_start(1, 1)
    l_copy(3, 1).start()

    def body(jj, carry):
        l = 2 * jj
        g_copy(0).wait()
        l_copy(l + 1, 1).wait()
        g_copy(1).start()
        w_wait(l - 2, 0)
        transpose(0)
        w_start(l, 0)
        l_copy(l + 2, 0).start()
        g_copy(1).wait()
        l_copy(l + 2, 0).wait()
        g_copy(0).start()
        w_wait(l - 1, 1)
        transpose(1)
        w_start(l + 1, 1)
        l_copy(l + 3, 1).start()
        return carry

    lax.fori_loop(1, _BPW // 2 - 1, body, 0)

    ll = _BPW - 2
    g_copy(0).wait()
    l_copy(ll + 1, 1).wait()
    g_copy(1).start()
    w_wait(ll - 2, 0)
    transpose(0)
    w_start(ll, 0)
    g_copy(1).wait()
    w_wait(ll - 1, 1)
    transpose(1)
    w_start(ll + 1, 1)
    w_wait(ll, 0)
    w_wait(ll + 1, 1)


def kernel(x, table):
    xt = jnp.transpose(x).reshape(-1).astype(jnp.int32)
    z = _gather_kernel(xt, table).reshape(_HIST, 4, _BATCH // _MB, 8, _MB)
    zt = lax.transpose(z, (2, 4, 0, 1, 3))
    return zt.reshape(_BATCH, _HIST, _EMBED)


# speedup vs baseline: 1.0652x; 1.0652x over previous
import functools

import jax
import jax.numpy as jnp
from jax import lax
from jax.experimental import pallas as pl
from jax.experimental.pallas import tpu as pltpu
from jax.experimental.pallas import tpu_sc as plsc

_BATCH = 16384
_HIST = 200
_EMBED = 32
_N = _BATCH * _HIST

_NC = 2
_NS = 16
_NW = _NC * _NS
_MB = 128
_SB = 512
_NSB = _N // _SB
_BPW = _NSB // _NW
_SBH = _BATCH // _SB

_mesh = plsc.VectorSubcoreMesh(core_axis_name="c", subcore_axis_name="s")


@functools.partial(
    pl.kernel,
    out_type=jax.ShapeDtypeStruct((_HIST, 4, _BATCH * 8), jnp.float32),
    mesh=_mesh,
    scratch_types=[
        pltpu.VMEM((_SB,), jnp.int32),
        pltpu.VMEM((_SB,), jnp.int32),
        pltpu.VMEM((_SB, _EMBED), jnp.float32),
        pltpu.VMEM((_SB, _EMBED), jnp.float32),
        pltpu.VMEM((4 * 4096,), jnp.float32),
        pltpu.VMEM((4 * 4096,), jnp.float32),
        pltpu.SemaphoreType.DMA,
        pltpu.SemaphoreType.DMA,
        pltpu.SemaphoreType.DMA,
        pltpu.SemaphoreType.DMA,
        pltpu.SemaphoreType.DMA,
        pltpu.SemaphoreType.DMA,
    ],
    compiler_params=pltpu.CompilerParams(use_tc_tiling_on_sc=False,
                                         needs_layout_passes=False),
)
def _gather_kernel(xt_hbm, table_hbm, out_hbm, idx_v0, idx_v1,
                   rows_v0, rows_v1, buf_v0, buf_v1,
                   sem_l0, sem_l1, sem_g0, sem_g1, sem_w0, sem_w1):
    wid = lax.axis_index("s") * _NC + lax.axis_index("c")
    t0 = wid * _BPW
    idx_v = (idx_v0, idx_v1)
    rows_v = (rows_v0, rows_v1)
    buf_v = (buf_v0, buf_v1)
    sem_l = (sem_l0, sem_l1)
    sem_g = (sem_g0, sem_g1)
    sem_w = (sem_w0, sem_w1)

    iota = lax.iota(jnp.int32, 16)

    def hB(l):
        t = t0 + l
        return t // _SBH, t % _SBH

    def l_copy(l, b):
        h, B = hB(l)
        return pltpu.make_async_copy(
            xt_hbm.at[pl.ds(h * _BATCH + B * _SB, _SB)], idx_v[b], sem_l[b])

    def g_copy(b):
        return pltpu.make_async_copy(
            table_hbm.at[idx_v[b]], rows_v[b], sem_g[b])

    def _w_copies(l, b):
        h, B = hB(l)
        return [
            pltpu.make_async_copy(
                buf_v[b].at[pl.ds(q * 4096, 4096)],
                out_hbm.at[h, q, pl.ds(B * 4096, 4096)], sem_w[b])
            for q in range(4)
        ]

    def w_start(l, b):
        for c in _w_copies(l, b):
            c.start()

    def w_wait(l, b):
        for c in _w_copies(l, b):
            c.wait()

    def transpose(b):
        def step(c0, carry):
            cm = (c0 + iota) & 31
            sbase = ((cm & 0x18) << 9) + ((cm & 7) << 7) + iota

            def load(k):
                return plsc.load_gather(rows_v[b], [iota + 16 * k, cm])

            def store(k, v):
                off_k = 1024 * (k // 8) + (16 * k % 128)
                plsc.store_scatter(buf_v[b], [sbase + off_k], v)

            depth = 6
            pend = [load(k) for k in range(depth)]
            for k in range(32 - depth):
                nxt = load(k + depth)
                store(k, pend[0])
                pend = pend[1:] + [nxt]
            for k in range(32 - depth, 32):
                store(k, pend[0])
                pend = pend[1:]
            return carry

        lax.fori_loop(0, 32, step, 0)

    l_copy(0, 0).start()
    l_copy(1, 1).start()
    l_copy(0, 0).wait()
    g_copy(0).start()
    g_copy(0).wait()
    l_copy(1, 1).wait()
    g_copy(1).start()
    transpose(0)
    w_start(0, 0)
    l_copy(2, 0).start()
    g_copy(1).wait()
    l_copy(2, 0).wait()
    g_copy(0).start()
    transpose(1)
    w_start(1, 1)
    l_copy(3, 1).start()

    def body(jj, carry):
        l = 2 * jj
        g_copy(0).wait()
        l_copy(l + 1, 1).wait()
        g_copy(1).start()
        w_wait(l - 2, 0)
        transpose(0)
        w_start(l, 0)
        l_copy(l + 2, 0).start()
        g_copy(1).wait()
        l_copy(l + 2, 0).wait()
        g_copy(0).start()
        w_wait(l - 1, 1)
        transpose(1)
        w_start(l + 1, 1)
        l_copy(l + 3, 1).start()
        return carry

    lax.fori_loop(1, _BPW // 2 - 1, body, 0)

    ll = _BPW - 2
    g_copy(0).wait()
    l_copy(ll + 1, 1).wait()
    g_copy(1).start()
    w_wait(ll - 2, 0)
    transpose(0)
    w_start(ll, 0)
    g_copy(1).wait()
    w_wait(ll - 1, 1)
    transpose(1)
    w_start(ll + 1, 1)
    w_wait(ll, 0)
    w_wait(ll + 1, 1)


def kernel(x, table):
    xt = jnp.transpose(x).reshape(-1).astype(jnp.int32)
    z = _gather_kernel(xt, table).reshape(_HIST, 4, _BATCH // _MB, 8, _MB)
    zt = lax.transpose(z, (2, 4, 0, 1, 3))
    return zt.reshape(_BATCH, _HIST, _EMBED)


# speedup vs baseline: 1.0865x; 1.0200x over previous
import functools

import jax
import jax.numpy as jnp
from jax import lax
from jax.experimental import pallas as pl
from jax.experimental.pallas import tpu as pltpu
from jax.experimental.pallas import tpu_sc as plsc

_BATCH = 16384
_HIST = 200
_EMBED = 32
_N = _BATCH * _HIST

_NC = 2
_NS = 16
_NW = _NC * _NS
_MB = 128
_SB = 512
_NSB = _N // _SB
_BPW = _NSB // _NW
_SBH = _BATCH // _SB

_mesh = plsc.VectorSubcoreMesh(core_axis_name="c", subcore_axis_name="s")


@functools.partial(
    pl.kernel,
    out_type=jax.ShapeDtypeStruct((_HIST, 4, _BATCH * 8), jnp.float32),
    mesh=_mesh,
    scratch_types=[
        pltpu.VMEM((_SB,), jnp.int32),
        pltpu.VMEM((_SB,), jnp.int32),
        pltpu.VMEM((_SB, _EMBED), jnp.float32),
        pltpu.VMEM((_SB, _EMBED), jnp.float32),
        pltpu.VMEM((4 * 4096,), jnp.float32),
        pltpu.VMEM((4 * 4096,), jnp.float32),
        pltpu.SemaphoreType.DMA,
        pltpu.SemaphoreType.DMA,
        pltpu.SemaphoreType.DMA,
        pltpu.SemaphoreType.DMA,
        pltpu.SemaphoreType.DMA,
        pltpu.SemaphoreType.DMA,
    ],
    compiler_params=pltpu.CompilerParams(use_tc_tiling_on_sc=False,
                                         needs_layout_passes=False),
)
def _gather_kernel(xt_hbm, table_hbm, out_hbm, idx_v0, idx_v1,
                   rows_v0, rows_v1, buf_v0, buf_v1,
                   sem_l0, sem_l1, sem_g0, sem_g1, sem_w0, sem_w1):
    wid = lax.axis_index("s") * _NC + lax.axis_index("c")
    t0 = wid * _BPW
    idx_v = (idx_v0, idx_v1)
    rows_v = (rows_v0, rows_v1)
    buf_v = (buf_v0, buf_v1)
    sem_l = (sem_l0, sem_l1)
    sem_g = (sem_g0, sem_g1)
    sem_w = (sem_w0, sem_w1)

    iota = lax.iota(jnp.int32, 16)

    def hB(l):
        t = t0 + l
        return t // _SBH, t % _SBH

    def l_copy(l, b):
        h, B = hB(l)
        return pltpu.make_async_copy(
            xt_hbm.at[pl.ds(h * _BATCH + B * _SB, _SB)], idx_v[b], sem_l[b])

    class _GGroup:
        def __init__(self, copies):
            self.copies = copies

        def start(self):
            for c in self.copies:
                c.start()

        def wait(self):
            for c in self.copies:
                c.wait()

    def g_copy(b):
        return _GGroup([
            pltpu.make_async_copy(
                table_hbm.at[idx_v[b].at[pl.ds(half * (_SB // 2), _SB // 2)]],
                rows_v[b].at[pl.ds(half * (_SB // 2), _SB // 2)], sem_g[b])
            for half in range(2)
        ])

    def _w_copies(l, b):
        h, B = hB(l)
        return [
            pltpu.make_async_copy(
                buf_v[b].at[pl.ds(q * 4096, 4096)],
                out_hbm.at[h, q, pl.ds(B * 4096, 4096)], sem_w[b])
            for q in range(4)
        ]

    def w_start(l, b):
        for c in _w_copies(l, b):
            c.start()

    def w_wait(l, b):
        for c in _w_copies(l, b):
            c.wait()

    def transpose(b):
        def step(c0, carry):
            cm = (c0 + iota) & 31
            sbase = ((cm & 0x18) << 9) + ((cm & 7) << 7) + iota

            def load(k):
                return plsc.load_gather(rows_v[b], [iota + 16 * k, cm])

            def store(k, v):
                off_k = 1024 * (k // 8) + (16 * k % 128)
                plsc.store_scatter(buf_v[b], [sbase + off_k], v)

            depth = 4
            pend = [load(k) for k in range(depth)]
            for k in range(32 - depth):
                nxt = load(k + depth)
                store(k, pend[0])
                pend = pend[1:] + [nxt]
            for k in range(32 - depth, 32):
                store(k, pend[0])
                pend = pend[1:]
            return carry

        lax.fori_loop(0, 32, step, 0)

    l_copy(0, 0).start()
    l_copy(1, 1).start()
    l_copy(0, 0).wait()
    g_copy(0).start()
    g_copy(0).wait()
    l_copy(1, 1).wait()
    g_copy(1).start()
    transpose(0)
    w_start(0, 0)
    l_copy(2, 0).start()
    g_copy(1).wait()
    l_copy(2, 0).wait()
    g_copy(0).start()
    transpose(1)
    w_start(1, 1)
    l_copy(3, 1).start()

    def body(jj, carry):
        l = 2 * jj
        g_copy(0).wait()
        l_copy(l + 1, 1).wait()
        g_copy(1).start()
        w_wait(l - 2, 0)
        transpose(0)
        w_start(l, 0)
        l_copy(l + 2, 0).start()
        g_copy(1).wait()
        l_copy(l + 2, 0).wait()
        g_copy(0).start()
        w_wait(l - 1, 1)
        transpose(1)
        w_start(l + 1, 1)
        l_copy(l + 3, 1).start()
        return carry

    lax.fori_loop(1, _BPW // 2 - 1, body, 0)

    ll = _BPW - 2
    g_copy(0).wait()
    l_copy(ll + 1, 1).wait()
    g_copy(1).start()
    w_wait(ll - 2, 0)
    transpose(0)
    w_start(ll, 0)
    g_copy(1).wait()
    w_wait(ll - 1, 1)
    transpose(1)
    w_start(ll + 1, 1)
    w_wait(ll, 0)
    w_wait(ll + 1, 1)


def kernel(x, table):
    xt = jnp.transpose(x).reshape(-1).astype(jnp.int32)
    z = _gather_kernel(xt, table).reshape(_HIST, 4, _BATCH // _MB, 8, _MB)
    zt = lax.transpose(z, (2, 4, 0, 1, 3))
    return zt.reshape(_BATCH, _HIST, _EMBED)
